# baseline (speedup 1.0000x reference)
import jax
import jax.numpy as jnp
from jax.experimental import pallas as pl
from jax.experimental.pallas import tpu as pltpu

_B = 64
_GB = 64
_XS = 72
_OS = 40


def _gconv_kernel(x_ref, w_ref, b_ref, o_ref, xt_ref, ot_ref):
    for b in range(_B):
        v = x_ref[b]
        xt_ref[0, pl.ds(b, _GB, stride=_XS), :] = v[:, :128]
        xt_ref[1, pl.ds(b, _GB, stride=_XS), :] = v[:, 128:]
    for g in range(_GB):
        r = pl.ds(g * _XS, _B)
        lhs = jnp.concatenate(
            [xt_ref[0, r, :], xt_ref[1, r, :]], axis=1)
        og = jax.lax.dot_general(
            lhs, w_ref[g],
            dimension_numbers=(((1,), (1,)), ((), ())),
            preferred_element_type=jnp.float32,
        ) + b_ref[g : g + 1, :]
        ro = pl.ds(g, _B, stride=_OS)
        ot_ref[0, ro, :] = og[:, :128]
        ot_ref[1, ro, :] = og[:, 128:]
    for b in range(_B):
        rq = pl.ds(b * _OS, _GB)
        o_ref[b] = jnp.concatenate(
            [ot_ref[0, rq, :], ot_ref[1, rq, :]], axis=1)


def kernel(x, W, b):
    B, Cin, H, Wsp = x.shape
    G = H * Wsp
    Cout = W.shape[1]
    xp = jnp.transpose(x, (0, 2, 3, 1)).reshape(B, G, Cin)
    out = pl.pallas_call(
        _gconv_kernel,
        grid=(G // _GB,),
        in_specs=[
            pl.BlockSpec((B, _GB, Cin), lambda j: (0, j, 0)),
            pl.BlockSpec((_GB, Cout, Cin), lambda j: (j, 0, 0)),
            pl.BlockSpec((_GB, Cout), lambda j: (j, 0)),
        ],
        out_specs=pl.BlockSpec((B, _GB, Cout), lambda j: (0, j, 0)),
        out_shape=jax.ShapeDtypeStruct((B, G, Cout), jnp.float32),
        scratch_shapes=[
            pltpu.VMEM((2, (_GB - 1) * _XS + _B, 128), jnp.float32),
            pltpu.VMEM((2, (_B - 1) * _OS + _GB, 128), jnp.float32),
        ],
        compiler_params=pltpu.CompilerParams(
            dimension_semantics=("parallel",),
            vmem_limit_bytes=60000 * 1024,
        ),
        name="regroup_conv_cl",
    )(xp, W, b)
    return jnp.transpose(out.reshape(B, H, Wsp, Cout), (0, 3, 1, 2))


# speedup vs baseline: 1.0021x; 1.0021x over previous
import jax
import jax.numpy as jnp
from jax.experimental import pallas as pl
from jax.experimental.pallas import tpu as pltpu

_B = 64
_GB = 32
_XS = 72
_OS = 40


def _gconv_kernel(x_ref, w_ref, b_ref, o_ref, xt_ref, ot_ref):
    for b in range(_B):
        v = x_ref[b]
        xt_ref[0, pl.ds(b, _GB, stride=_XS), :] = v[:, :128]
        xt_ref[1, pl.ds(b, _GB, stride=_XS), :] = v[:, 128:]
    for g in range(_GB):
        r = pl.ds(g * _XS, _B)
        lhs = jnp.concatenate(
            [xt_ref[0, r, :], xt_ref[1, r, :]], axis=1)
        og = jax.lax.dot_general(
            lhs, w_ref[g],
            dimension_numbers=(((1,), (1,)), ((), ())),
            preferred_element_type=jnp.float32,
        ) + b_ref[g : g + 1, :]
        ro = pl.ds(g, _B, stride=_OS)
        ot_ref[0, ro, :] = og[:, :128]
        ot_ref[1, ro, :] = og[:, 128:]
    for b in range(_B):
        rq = pl.ds(b * _OS, _GB)
        o_ref[b] = jnp.concatenate(
            [ot_ref[0, rq, :], ot_ref[1, rq, :]], axis=1)


def kernel(x, W, b):
    B, Cin, H, Wsp = x.shape
    G = H * Wsp
    Cout = W.shape[1]
    xp = jnp.transpose(x, (0, 2, 3, 1)).reshape(B, G, Cin)
    out = pl.pallas_call(
        _gconv_kernel,
        grid=(G // _GB,),
        in_specs=[
            pl.BlockSpec((B, _GB, Cin), lambda j: (0, j, 0)),
            pl.BlockSpec((_GB, Cout, Cin), lambda j: (j, 0, 0)),
            pl.BlockSpec((_GB, Cout), lambda j: (j, 0)),
        ],
        out_specs=pl.BlockSpec((B, _GB, Cout), lambda j: (0, j, 0)),
        out_shape=jax.ShapeDtypeStruct((B, G, Cout), jnp.float32),
        scratch_shapes=[
            pltpu.VMEM((2, (_GB - 1) * _XS + _B, 128), jnp.float32),
            pltpu.VMEM((2, (_B - 1) * _OS + _GB, 128), jnp.float32),
        ],
        compiler_params=pltpu.CompilerParams(
            dimension_semantics=("parallel",),
            vmem_limit_bytes=60000 * 1024,
        ),
        name="regroup_conv_cl",
    )(xp, W, b)
    return jnp.transpose(out.reshape(B, H, Wsp, Cout), (0, 3, 1, 2))
